# baseline (speedup 1.0000x reference)
import functools

import jax
import jax.numpy as jnp
from jax.experimental import pallas as pl

_BN = 512


def _mm_kernel(x_ref, w_ref, b_ref, o_ref):
    x = x_ref[...]
    w = w_ref[...].astype(jnp.bfloat16)
    acc = jax.lax.dot_general(
        x, w,
        dimension_numbers=(((1,), (1,)), ((), ())),
        preferred_element_type=jnp.float32,
    )
    o_ref[...] = acc + b_ref[...]


@functools.partial(jax.jit, static_argnames=())
def kernel(input, sparse_weight, bias):
    b, m, k = input.shape
    n = sparse_weight.shape[0]
    x = input.reshape(m, k).astype(jnp.bfloat16)
    bias2 = bias.reshape(1, n)
    out = pl.pallas_call(
        _mm_kernel,
        grid=(n // _BN,),
        in_specs=[
            pl.BlockSpec((m, k), lambda j: (0, 0)),
            pl.BlockSpec((_BN, k), lambda j: (j, 0)),
            pl.BlockSpec((1, _BN), lambda j: (0, j)),
        ],
        out_specs=pl.BlockSpec((m, _BN), lambda j: (0, j)),
        out_shape=jax.ShapeDtypeStruct((m, n), jnp.float32),
    )(x, sparse_weight, bias2)
    return out.reshape(b, m, n)


# speedup vs baseline: 1.1324x; 1.1324x over previous
import functools

import jax
import jax.numpy as jnp
from jax.experimental import pallas as pl
from jax.experimental.pallas import tpu as pltpu

_BN = 256


def _mm_kernel(x_ref, w_ref, b_ref, o_ref):
    acc = jax.lax.dot_general(
        x_ref[...], w_ref[...],
        dimension_numbers=(((1,), (1,)), ((), ())),
        precision=jax.lax.Precision.DEFAULT,
        preferred_element_type=jnp.float32,
    )
    o_ref[...] = acc + b_ref[...]


@functools.partial(jax.jit, static_argnames=())
def kernel(input, sparse_weight, bias):
    b, m, k = input.shape
    n = sparse_weight.shape[0]
    x = input.reshape(m, k)
    bias2 = bias.reshape(1, n)
    out = pl.pallas_call(
        _mm_kernel,
        grid=(n // _BN,),
        in_specs=[
            pl.BlockSpec((m, k), lambda j: (0, 0)),
            pl.BlockSpec((_BN, k), lambda j: (j, 0)),
            pl.BlockSpec((1, _BN), lambda j: (0, j)),
        ],
        out_specs=pl.BlockSpec((m, _BN), lambda j: (0, j)),
        out_shape=jax.ShapeDtypeStruct((m, n), jnp.float32),
        compiler_params=pltpu.CompilerParams(vmem_limit_bytes=62 * 1024 * 1024),
    )(x, sparse_weight, bias2)
    return out.reshape(b, m, n)


# speedup vs baseline: 1.1468x; 1.0127x over previous
import functools

import jax
import jax.numpy as jnp
from jax.experimental import pallas as pl
from jax.experimental.pallas import tpu as pltpu

_BN = 512


def _mm_kernel(x_ref, w_ref, b_ref, o_ref):
    acc = jax.lax.dot_general(
        x_ref[...], w_ref[...],
        dimension_numbers=(((1,), (1,)), ((), ())),
        precision=jax.lax.Precision.DEFAULT,
        preferred_element_type=jnp.float32,
    )
    o_ref[...] = acc + b_ref[...]


@functools.partial(jax.jit, static_argnames=())
def kernel(input, sparse_weight, bias):
    b, m, k = input.shape
    n = sparse_weight.shape[0]
    x = input.reshape(m, k)
    bias2 = bias.reshape(1, n)
    out = pl.pallas_call(
        _mm_kernel,
        grid=(n // _BN,),
        in_specs=[
            pl.BlockSpec((m, k), lambda j: (0, 0)),
            pl.BlockSpec((_BN, k), lambda j: (j, 0)),
            pl.BlockSpec((1, _BN), lambda j: (0, j)),
        ],
        out_specs=pl.BlockSpec((m, _BN), lambda j: (0, j)),
        out_shape=jax.ShapeDtypeStruct((m, n), jnp.float32),
        compiler_params=pltpu.CompilerParams(vmem_limit_bytes=62 * 1024 * 1024),
    )(x, sparse_weight, bias2)
    return out.reshape(b, m, n)


# speedup vs baseline: 1.2325x; 1.0748x over previous
import functools

import jax
import jax.numpy as jnp
from jax.experimental import pallas as pl
from jax.experimental.pallas import tpu as pltpu

_BN = 512
_NCHUNK = 8


def _dot(x, w, b):
    acc = jax.lax.dot_general(
        x, w,
        dimension_numbers=(((1,), (1,)), ((), ())),
        precision=jax.lax.Precision.DEFAULT,
        preferred_element_type=jnp.float32,
    )
    return acc + b


def _mm_kernel(x_hbm, w_ref, b_ref, o_ref, xv_ref, sems):
    m = xv_ref.shape[0]
    cm = m // _NCHUNK

    @pl.when(pl.program_id(0) == 0)
    def _first_step():
        for i in range(_NCHUNK):
            pltpu.make_async_copy(
                x_hbm.at[pl.ds(i * cm, cm), :],
                xv_ref.at[pl.ds(i * cm, cm), :],
                sems.at[i],
            ).start()
        for i in range(_NCHUNK):
            pltpu.make_async_copy(
                x_hbm.at[pl.ds(i * cm, cm), :],
                xv_ref.at[pl.ds(i * cm, cm), :],
                sems.at[i],
            ).wait()
            o_ref[pl.ds(i * cm, cm), :] = _dot(
                xv_ref[pl.ds(i * cm, cm), :], w_ref[...], b_ref[...])

    @pl.when(pl.program_id(0) != 0)
    def _rest():
        o_ref[...] = _dot(xv_ref[...], w_ref[...], b_ref[...])


@functools.partial(jax.jit, static_argnames=())
def kernel(input, sparse_weight, bias):
    b, m, k = input.shape
    n = sparse_weight.shape[0]
    x = input.reshape(m, k)
    bias2 = bias.reshape(1, n)
    out = pl.pallas_call(
        _mm_kernel,
        grid=(n // _BN,),
        in_specs=[
            pl.BlockSpec(memory_space=pl.ANY),
            pl.BlockSpec((_BN, k), lambda j: (j, 0)),
            pl.BlockSpec((1, _BN), lambda j: (0, j)),
        ],
        out_specs=pl.BlockSpec((m, _BN), lambda j: (0, j)),
        out_shape=jax.ShapeDtypeStruct((m, n), jnp.float32),
        scratch_shapes=[
            pltpu.VMEM((m, k), jnp.float32),
            pltpu.SemaphoreType.DMA((_NCHUNK,)),
        ],
        compiler_params=pltpu.CompilerParams(
            vmem_limit_bytes=62 * 1024 * 1024),
    )(x, sparse_weight, bias2)
    return out.reshape(b, m, n)
